# Initial kernel scaffold; baseline (speedup 1.0000x reference)
import jax
import jax.numpy as jnp
from jax.experimental import pallas as pl


def kernel(feat, edge_index, W_theta, b_theta, W_phi, b_phi):
    raise NotImplementedError("write your pallas kernel here")


# speedup vs baseline: 1.9749x; 1.9749x over previous
import functools

import jax
import jax.numpy as jnp
from jax import lax
from jax.experimental import pallas as pl
from jax.experimental.pallas import tpu as pltpu
from jax.experimental.pallas import tpu_sc as plsc

N = 10000
E = 320000
D = 128
NW = 32
CPW = D // NW
CHUNK = 10000
N_CHUNKS = E // CHUNK
GROUPS = CHUNK // 16


def _ab_body(feat_ref, wt_ref, wp_ref, bt_ref, bp_ref, a_ref, b_ref):
    x = feat_ref[...]
    wt = wt_ref[...]
    a_ref[...] = jnp.dot(x, wt, preferred_element_type=jnp.float32)
    b_ref[...] = (
        jnp.dot(x, wp_ref[...] - wt, preferred_element_type=jnp.float32)
        + bt_ref[...] + bp_ref[...]
    )


def _compute_ab(feat, wtT, wpT, bt, bp):
    grid = 10
    blk = N // grid
    return pl.pallas_call(
        _ab_body,
        grid=(grid,),
        in_specs=[
            pl.BlockSpec((blk, D), lambda i: (i, 0)),
            pl.BlockSpec((D, D), lambda i: (0, 0)),
            pl.BlockSpec((D, D), lambda i: (0, 0)),
            pl.BlockSpec((1, D), lambda i: (0, 0)),
            pl.BlockSpec((1, D), lambda i: (0, 0)),
        ],
        out_specs=[
            pl.BlockSpec((blk, D), lambda i: (i, 0)),
            pl.BlockSpec((blk, D), lambda i: (i, 0)),
        ],
        out_shape=[jax.ShapeDtypeStruct((N, D), jnp.float32)] * 2,
    )(feat, wtT, wpT, bt, bp)


def _sc_body(b_hbm, src_hbm, dst_hbm, m_hbm, bv, mv, sv, dv):
    cid = lax.axis_index("c")
    sid = lax.axis_index("s")
    wid = sid * 2 + cid

    pltpu.sync_copy(b_hbm.at[wid], bv)

    minf = jnp.full((16,), -jnp.inf, jnp.float32)

    def init(i, carry):
        mv[pl.ds(i * 16, 16)] = minf
        return carry

    lax.fori_loop(0, (N * CPW) // 16, init, 0)

    def chunk_body(k, carry):
        pltpu.sync_copy(src_hbm.at[pl.ds(k * CHUNK, CHUNK)], sv)
        pltpu.sync_copy(dst_hbm.at[pl.ds(k * CHUNK, CHUNK)], dv)

        def group(g, c2):
            s16 = sv[pl.ds(g * 16, 16)]
            d16 = dv[pl.ds(g * 16, 16)]
            sb = s16 * CPW
            db = d16 * CPW
            vals = [plsc.load_gather(bv, [sb + c]) for c in range(CPW)]

            cnt, _ = plsc.scan_count(d16)
            r_lo = jnp.min(cnt)
            r_hi = jnp.max(cnt)

            def round_body(st):
                r = st
                m = cnt == r
                for c in range(CPW):
                    cur = plsc.load_gather(mv, [db + c], mask=m)
                    plsc.store_scatter(
                        mv, [db + c], jnp.maximum(cur, vals[c]), mask=m
                    )
                return r + 1

            lax.while_loop(lambda r: r <= r_hi, round_body, r_lo)
            return c2

        lax.fori_loop(0, GROUPS, group, carry)
        return carry

    lax.fori_loop(0, N_CHUNKS, chunk_body, 0)

    pltpu.sync_copy(mv, m_hbm.at[wid])


def _sc_scatter_max(b_r, src, dst):
    mesh = plsc.VectorSubcoreMesh(core_axis_name="c", subcore_axis_name="s")
    f = functools.partial(
        pl.kernel,
        out_type=jax.ShapeDtypeStruct((NW, N * CPW), jnp.float32),
        mesh=mesh,
        scratch_types=[
            pltpu.VMEM((N * CPW,), jnp.float32),
            pltpu.VMEM((N * CPW,), jnp.float32),
            pltpu.VMEM((CHUNK,), jnp.int32),
            pltpu.VMEM((CHUNK,), jnp.int32),
        ],
        compiler_params=pltpu.CompilerParams(needs_layout_passes=False),
    )(_sc_body)
    return f(b_r, src, dst)


def _combine_body(a_ref, m_ref, o_ref):
    m = m_ref[...]
    o_ref[...] = jnp.where(jnp.isfinite(m), a_ref[...] + m, 0.0)


def _combine(a, m):
    grid = 10
    blk = N // grid
    return pl.pallas_call(
        _combine_body,
        grid=(grid,),
        in_specs=[
            pl.BlockSpec((blk, D), lambda i: (i, 0)),
            pl.BlockSpec((blk, D), lambda i: (i, 0)),
        ],
        out_specs=pl.BlockSpec((blk, D), lambda i: (i, 0)),
        out_shape=jax.ShapeDtypeStruct((N, D), jnp.float32),
    )(a, m)


def kernel(feat, edge_index, W_theta, b_theta, W_phi, b_phi):
    src = edge_index[0].astype(jnp.int32)
    dst = edge_index[1].astype(jnp.int32)
    a, b = _compute_ab(
        feat,
        W_theta.T,
        W_phi.T,
        b_theta.reshape(1, D),
        b_phi.reshape(1, D),
    )
    b_r = b.reshape(N, NW, CPW).transpose(1, 0, 2).reshape(NW, N * CPW)
    m_r = _sc_scatter_max(b_r, src, dst)
    m = m_r.reshape(NW, N, CPW).transpose(1, 0, 2).reshape(N, D)
    return _combine(a, m)


# speedup vs baseline: 3.2209x; 1.6309x over previous
import functools

import jax
import jax.numpy as jnp
from jax import lax
from jax.experimental import pallas as pl
from jax.experimental.pallas import tpu as pltpu
from jax.experimental.pallas import tpu_sc as plsc

N = 10000
E = 320000
D = 128
NW = 32
CPW = D // NW
CHUNK = 10000
N_CHUNKS = E // CHUNK
GROUPS = CHUNK // 16


def _ab_body(feat_ref, wt_ref, wp_ref, bt_ref, bp_ref, a_ref, b_ref):
    x = feat_ref[...]
    wt = wt_ref[...]
    a_ref[...] = jnp.dot(x, wt, preferred_element_type=jnp.float32)
    b_ref[...] = (
        jnp.dot(x, wp_ref[...] - wt, preferred_element_type=jnp.float32)
        + bt_ref[...] + bp_ref[...]
    )


def _compute_ab(feat, wtT, wpT, bt, bp):
    grid = 10
    blk = N // grid
    return pl.pallas_call(
        _ab_body,
        grid=(grid,),
        in_specs=[
            pl.BlockSpec((blk, D), lambda i: (i, 0)),
            pl.BlockSpec((D, D), lambda i: (0, 0)),
            pl.BlockSpec((D, D), lambda i: (0, 0)),
            pl.BlockSpec((1, D), lambda i: (0, 0)),
            pl.BlockSpec((1, D), lambda i: (0, 0)),
        ],
        out_specs=[
            pl.BlockSpec((blk, D), lambda i: (i, 0)),
            pl.BlockSpec((blk, D), lambda i: (i, 0)),
        ],
        out_shape=[jax.ShapeDtypeStruct((N, D), jnp.float32)] * 2,
    )(feat, wtT, wpT, bt, bp)


def _sc_body(b_hbm, src_hbm, dst_hbm, m_hbm,
             bv0, bv1, bv2, bv3, mv0, mv1, mv2, mv3, sv, dv):
    cid = lax.axis_index("c")
    sid = lax.axis_index("s")
    wid = sid * 2 + cid
    bvs = [bv0, bv1, bv2, bv3]
    mvs = [mv0, mv1, mv2, mv3]

    for c in range(CPW):
        pltpu.sync_copy(b_hbm.at[wid, c], bvs[c])

    minf = jnp.full((16,), -jnp.inf, jnp.float32)

    def init(i, carry):
        for c in range(CPW):
            mvs[c][pl.ds(i * 16, 16)] = minf
        return carry

    lax.fori_loop(0, N // 16, init, 0, unroll=4)

    iota = lax.iota(jnp.int32, 16)

    def chunk_body(k, carry):
        pltpu.sync_copy(src_hbm.at[pl.ds(k * CHUNK, CHUNK)], sv)
        pltpu.sync_copy(dst_hbm.at[pl.ds(k * CHUNK, CHUNK)], dv)

        def group(g, c2):
            s16 = sv[pl.ds(g * 16, 16)]
            d16 = dv[pl.ds(g * 16, 16)]
            vals = [plsc.load_gather(bvs[c], [s16]) for c in range(CPW)]

            _, last = plsc.scan_count(d16)
            for c in range(CPW):
                cur = plsc.load_gather(mvs[c], [d16], mask=last)
                plsc.store_scatter(
                    mvs[c], [d16], jnp.maximum(cur, vals[c]), mask=last
                )

            rem = jnp.logical_not(last)

            @pl.when(jnp.any(rem))
            def _slow():
                def lane(j, cc):
                    m = jnp.logical_and(rem, iota == j)
                    for c in range(CPW):
                        cur = plsc.load_gather(mvs[c], [d16], mask=m)
                        plsc.store_scatter(
                            mvs[c], [d16], jnp.maximum(cur, vals[c]), mask=m
                        )
                    return cc

                lax.fori_loop(0, 16, lane, 0)

            return c2

        lax.fori_loop(0, GROUPS, group, carry, unroll=4)
        return carry

    lax.fori_loop(0, N_CHUNKS, chunk_body, 0)

    for c in range(CPW):
        pltpu.sync_copy(mvs[c], m_hbm.at[wid, c])


def _sc_scatter_max(b_r, src, dst):
    mesh = plsc.VectorSubcoreMesh(core_axis_name="c", subcore_axis_name="s")
    f = functools.partial(
        pl.kernel,
        out_type=jax.ShapeDtypeStruct((NW, CPW, N), jnp.float32),
        mesh=mesh,
        scratch_types=(
            [pltpu.VMEM((N,), jnp.float32) for _ in range(2 * CPW)]
            + [pltpu.VMEM((CHUNK,), jnp.int32) for _ in range(2)]
        ),
        compiler_params=pltpu.CompilerParams(needs_layout_passes=False),
    )(_sc_body)
    return f(b_r, src, dst)


def _combine_body(a_ref, m_ref, o_ref):
    m = m_ref[...]
    o_ref[...] = jnp.where(jnp.isfinite(m), a_ref[...] + m, 0.0)


def _combine(a, m):
    grid = 10
    blk = N // grid
    return pl.pallas_call(
        _combine_body,
        grid=(grid,),
        in_specs=[
            pl.BlockSpec((blk, D), lambda i: (i, 0)),
            pl.BlockSpec((blk, D), lambda i: (i, 0)),
        ],
        out_specs=pl.BlockSpec((blk, D), lambda i: (i, 0)),
        out_shape=jax.ShapeDtypeStruct((N, D), jnp.float32),
    )(a, m)


def kernel(feat, edge_index, W_theta, b_theta, W_phi, b_phi):
    src = edge_index[0].astype(jnp.int32)
    dst = edge_index[1].astype(jnp.int32)
    a, b = _compute_ab(
        feat,
        W_theta.T,
        W_phi.T,
        b_theta.reshape(1, D),
        b_phi.reshape(1, D),
    )
    b_r = b.reshape(N, NW, CPW).transpose(1, 2, 0)
    m_r = _sc_scatter_max(b_r, src, dst)
    m = m_r.transpose(2, 0, 1).reshape(N, D)
    return _combine(a, m)


# speedup vs baseline: 4.1270x; 1.2813x over previous
import functools

import jax
import jax.numpy as jnp
from jax import lax
from jax.experimental import pallas as pl
from jax.experimental.pallas import tpu as pltpu
from jax.experimental.pallas import tpu_sc as plsc

N = 10000
E = 320000
D = 128
NW = 32
CPW = D // NW
CHUNK = 16000
N_CHUNKS = E // CHUNK
GROUPS = CHUNK // 16
NB = 4


def _ab_body(feat_ref, wt_ref, wp_ref, bt_ref, bp_ref, a_ref, b_ref):
    x = feat_ref[...]
    wt = wt_ref[...]
    a_ref[...] = jnp.dot(x, wt, preferred_element_type=jnp.float32)
    b_ref[...] = (
        jnp.dot(x, wp_ref[...] - wt, preferred_element_type=jnp.float32)
        + bt_ref[...] + bp_ref[...]
    )


def _compute_ab(feat, wtT, wpT, bt, bp):
    grid = 10
    blk = N // grid
    return pl.pallas_call(
        _ab_body,
        grid=(grid,),
        in_specs=[
            pl.BlockSpec((blk, D), lambda i: (i, 0)),
            pl.BlockSpec((D, D), lambda i: (0, 0)),
            pl.BlockSpec((D, D), lambda i: (0, 0)),
            pl.BlockSpec((1, D), lambda i: (0, 0)),
            pl.BlockSpec((1, D), lambda i: (0, 0)),
        ],
        out_specs=[
            pl.BlockSpec((blk, D), lambda i: (i, 0)),
            pl.BlockSpec((blk, D), lambda i: (i, 0)),
        ],
        out_shape=[jax.ShapeDtypeStruct((N, D), jnp.float32)] * 2,
    )(feat, wtT, wpT, bt, bp)


def _sc_body(b_hbm, src_hbm, dst_hbm, m_hbm,
             bv0, bv1, bv2, bv3, mv0, mv1, mv2, mv3, sv, dv):
    cid = lax.axis_index("c")
    sid = lax.axis_index("s")
    wid = sid * 2 + cid
    bvs = [bv0, bv1, bv2, bv3]
    mvs = [mv0, mv1, mv2, mv3]

    for c in range(CPW):
        pltpu.sync_copy(b_hbm.at[wid, c], bvs[c])

    minf = jnp.full((16,), -jnp.inf, jnp.float32)

    def init(i, carry):
        for c in range(CPW):
            mvs[c][pl.ds(i * 16, 16)] = minf
        return carry

    lax.fori_loop(0, N // 16, init, 0, unroll=4)

    iota = lax.iota(jnp.int32, 16)

    def chunk_body(k, carry):
        pltpu.sync_copy(src_hbm.at[pl.ds(k * CHUNK, CHUNK)], sv)
        pltpu.sync_copy(dst_hbm.at[pl.ds(k * CHUNK, CHUNK)], dv)

        def block(t, c2):
            base = t * (16 * NB)
            d16s = [dv[pl.ds(base + 16 * i, 16)] for i in range(NB)]
            s16s = [sv[pl.ds(base + 16 * i, 16)] for i in range(NB)]
            valss = [
                [plsc.load_gather(bvs[c], [s16s[i]]) for c in range(CPW)]
                for i in range(NB)
            ]
            lasts = [plsc.scan_count(d16s[i])[1] for i in range(NB)]
            for i in range(NB):
                curs = [
                    plsc.load_gather(mvs[c], [d16s[i]]) for c in range(CPW)
                ]
                news = [jnp.maximum(curs[c], valss[i][c]) for c in range(CPW)]
                for c in range(CPW):
                    plsc.store_scatter(
                        mvs[c], [d16s[i]], news[c], mask=lasts[i]
                    )

            rems = [jnp.logical_not(lasts[i]) for i in range(NB)]
            any_rem = rems[0]
            for i in range(1, NB):
                any_rem = jnp.logical_or(any_rem, rems[i])

            @pl.when(jnp.any(any_rem))
            def _slow():
                for i in range(NB):
                    def lane(j, cc, i=i):
                        m = jnp.logical_and(rems[i], iota == j)
                        for c in range(CPW):
                            cur = plsc.load_gather(mvs[c], [d16s[i]], mask=m)
                            plsc.store_scatter(
                                mvs[c], [d16s[i]],
                                jnp.maximum(cur, valss[i][c]), mask=m,
                            )
                        return cc

                    lax.fori_loop(0, 16, lane, 0)

            return c2

        lax.fori_loop(0, GROUPS // NB, block, carry)
        return carry

    lax.fori_loop(0, N_CHUNKS, chunk_body, 0)

    for c in range(CPW):
        pltpu.sync_copy(mvs[c], m_hbm.at[wid, c])


def _sc_scatter_max(b_r, src, dst):
    mesh = plsc.VectorSubcoreMesh(core_axis_name="c", subcore_axis_name="s")
    f = functools.partial(
        pl.kernel,
        out_type=jax.ShapeDtypeStruct((NW, CPW, N), jnp.float32),
        mesh=mesh,
        scratch_types=(
            [pltpu.VMEM((N,), jnp.float32) for _ in range(2 * CPW)]
            + [pltpu.VMEM((CHUNK,), jnp.int32) for _ in range(2)]
        ),
        compiler_params=pltpu.CompilerParams(needs_layout_passes=False),
    )(_sc_body)
    return f(b_r, src, dst)


def _combine_body(a_ref, m_ref, o_ref):
    m = m_ref[...]
    o_ref[...] = jnp.where(jnp.isfinite(m), a_ref[...] + m, 0.0)


def _combine(a, m):
    grid = 10
    blk = N // grid
    return pl.pallas_call(
        _combine_body,
        grid=(grid,),
        in_specs=[
            pl.BlockSpec((blk, D), lambda i: (i, 0)),
            pl.BlockSpec((blk, D), lambda i: (i, 0)),
        ],
        out_specs=pl.BlockSpec((blk, D), lambda i: (i, 0)),
        out_shape=jax.ShapeDtypeStruct((N, D), jnp.float32),
    )(a, m)


def kernel(feat, edge_index, W_theta, b_theta, W_phi, b_phi):
    src = edge_index[0].astype(jnp.int32)
    dst = edge_index[1].astype(jnp.int32)
    a, b = _compute_ab(
        feat,
        W_theta.T,
        W_phi.T,
        b_theta.reshape(1, D),
        b_phi.reshape(1, D),
    )
    b_r = b.reshape(N, NW, CPW).transpose(1, 2, 0)
    m_r = _sc_scatter_max(b_r, src, dst)
    m = m_r.transpose(2, 0, 1).reshape(N, D)
    return _combine(a, m)


# speedup vs baseline: 7.5202x; 1.8222x over previous
import functools

import jax
import jax.numpy as jnp
from jax import lax
from jax.experimental import pallas as pl
from jax.experimental.pallas import tpu as pltpu
from jax.experimental.pallas import tpu_sc as plsc

N = 10000
E = 320000
D = 128
NW = 32
CPW = D // NW
CHUNK = 8000
N_CHUNKS = E // CHUNK
GROUPS = CHUNK // 16
NB = 4


def _ab_body(feat_ref, wt_ref, wp_ref, bt_ref, bp_ref, a_ref, b_ref):
    x = feat_ref[...]
    wt = wt_ref[...]
    a_ref[...] = jnp.dot(x, wt, preferred_element_type=jnp.float32)
    b_ref[...] = (
        jnp.dot(x, wp_ref[...] - wt, preferred_element_type=jnp.float32)
        + bt_ref[...] + bp_ref[...]
    )


def _compute_ab(feat, wtT, wpT, bt, bp):
    grid = 10
    blk = N // grid
    return pl.pallas_call(
        _ab_body,
        grid=(grid,),
        in_specs=[
            pl.BlockSpec((blk, D), lambda i: (i, 0)),
            pl.BlockSpec((D, D), lambda i: (0, 0)),
            pl.BlockSpec((D, D), lambda i: (0, 0)),
            pl.BlockSpec((1, D), lambda i: (0, 0)),
            pl.BlockSpec((1, D), lambda i: (0, 0)),
        ],
        out_specs=[
            pl.BlockSpec((blk, D), lambda i: (i, 0)),
            pl.BlockSpec((blk, D), lambda i: (i, 0)),
        ],
        out_shape=[jax.ShapeDtypeStruct((N, D), jnp.float32)] * 2,
    )(feat, wtT, wpT, bt, bp)


def _sc_body(b_hbm, src_hbm, dst_hbm, m_hbm,
             bv0, bv1, bv2, bv3, mv0, mv1, mv2, mv3,
             sv0, dv0, sv1, dv1, sem0, sem1):
    cid = lax.axis_index("c")
    sid = lax.axis_index("s")
    wid = sid * 2 + cid
    bvs = [bv0, bv1, bv2, bv3]
    mvs = [mv0, mv1, mv2, mv3]
    svs = [sv0, sv1]
    dvs = [dv0, dv1]
    sems = [sem0, sem1]

    def start_copy(k, buf):
        sl = pl.ds(k * CHUNK, CHUNK)
        pltpu.make_async_copy(src_hbm.at[sl], svs[buf], sems[buf]).start()
        pltpu.make_async_copy(dst_hbm.at[sl], dvs[buf], sems[buf]).start()

    def wait_copy(k, buf):
        sl = pl.ds(k * CHUNK, CHUNK)
        pltpu.make_async_copy(src_hbm.at[sl], svs[buf], sems[buf]).wait()
        pltpu.make_async_copy(dst_hbm.at[sl], dvs[buf], sems[buf]).wait()

    for c in range(CPW):
        pltpu.sync_copy(b_hbm.at[wid, c], bvs[c])

    minf = jnp.full((16,), -jnp.inf, jnp.float32)

    def init(i, carry):
        for c in range(CPW):
            mvs[c][pl.ds(i * 16, 16)] = minf
        return carry

    lax.fori_loop(0, N // 16, init, 0, unroll=4)

    iota = lax.iota(jnp.int32, 16)

    def process(sv, dv):
        def block(t, c2):
            base = t * (16 * NB)
            d16s = [dv[pl.ds(base + 16 * i, 16)] for i in range(NB)]
            s16s = [sv[pl.ds(base + 16 * i, 16)] for i in range(NB)]
            valss = [
                [plsc.load_gather(bvs[c], [s16s[i]]) for c in range(CPW)]
                for i in range(NB)
            ]
            lasts = [plsc.scan_count(d16s[i])[1] for i in range(NB)]
            for i in range(NB):
                curs = [
                    plsc.load_gather(mvs[c], [d16s[i]]) for c in range(CPW)
                ]
                news = [jnp.maximum(curs[c], valss[i][c]) for c in range(CPW)]
                for c in range(CPW):
                    plsc.store_scatter(
                        mvs[c], [d16s[i]], news[c], mask=lasts[i]
                    )

            rems = [jnp.logical_not(lasts[i]) for i in range(NB)]
            any_rem = rems[0]
            for i in range(1, NB):
                any_rem = jnp.logical_or(any_rem, rems[i])

            @pl.when(jnp.any(any_rem))
            def _slow():
                for i in range(NB):
                    @pl.when(jnp.any(rems[i]))
                    def _g(i=i):
                        def round_(rem):
                            _, last2 = plsc.scan_count(d16s[i], mask=rem)
                            for c in range(CPW):
                                cur = plsc.load_gather(mvs[c], [d16s[i]])
                                plsc.store_scatter(
                                    mvs[c], [d16s[i]],
                                    jnp.maximum(cur, valss[i][c]), mask=last2,
                                )
                            return jnp.logical_and(
                                rem, jnp.logical_not(last2)
                            )

                        lax.while_loop(
                            lambda r: jnp.any(r), round_, rems[i]
                        )

            return c2

        lax.fori_loop(0, GROUPS // NB, block, 0)

    start_copy(0, 0)

    def pair(p, carry):
        start_copy(2 * p + 1, 1)
        wait_copy(2 * p, 0)
        process(svs[0], dvs[0])

        @pl.when(p < N_CHUNKS // 2 - 1)
        def _prefetch():
            start_copy(2 * p + 2, 0)

        wait_copy(2 * p + 1, 1)
        process(svs[1], dvs[1])
        return carry

    lax.fori_loop(0, N_CHUNKS // 2, pair, 0)

    for c in range(CPW):
        pltpu.sync_copy(mvs[c], m_hbm.at[wid, c])


def _sc_scatter_max(b_r, src, dst):
    mesh = plsc.VectorSubcoreMesh(core_axis_name="c", subcore_axis_name="s")
    f = functools.partial(
        pl.kernel,
        out_type=jax.ShapeDtypeStruct((NW, CPW, N), jnp.float32),
        mesh=mesh,
        scratch_types=(
            [pltpu.VMEM((N,), jnp.float32) for _ in range(2 * CPW)]
            + [pltpu.VMEM((CHUNK,), jnp.int32) for _ in range(4)]
            + [pltpu.SemaphoreType.DMA for _ in range(2)]
        ),
        compiler_params=pltpu.CompilerParams(needs_layout_passes=False),
    )(_sc_body)
    return f(b_r, src, dst)


def _combine_body(a_ref, m_ref, o_ref):
    m = m_ref[...]
    o_ref[...] = jnp.where(jnp.isfinite(m), a_ref[...] + m, 0.0)


def _combine(a, m):
    grid = 10
    blk = N // grid
    return pl.pallas_call(
        _combine_body,
        grid=(grid,),
        in_specs=[
            pl.BlockSpec((blk, D), lambda i: (i, 0)),
            pl.BlockSpec((blk, D), lambda i: (i, 0)),
        ],
        out_specs=pl.BlockSpec((blk, D), lambda i: (i, 0)),
        out_shape=jax.ShapeDtypeStruct((N, D), jnp.float32),
    )(a, m)


def kernel(feat, edge_index, W_theta, b_theta, W_phi, b_phi):
    src = edge_index[0].astype(jnp.int32)
    dst = edge_index[1].astype(jnp.int32)
    a, b = _compute_ab(
        feat,
        W_theta.T,
        W_phi.T,
        b_theta.reshape(1, D),
        b_phi.reshape(1, D),
    )
    b_r = b.reshape(N, NW, CPW).transpose(1, 2, 0)
    m_r = _sc_scatter_max(b_r, src, dst)
    m = m_r.transpose(2, 0, 1).reshape(N, D)
    return _combine(a, m)


# speedup vs baseline: 8.5709x; 1.1397x over previous
import functools

import jax
import jax.numpy as jnp
from jax import lax
from jax.experimental import pallas as pl
from jax.experimental.pallas import tpu as pltpu
from jax.experimental.pallas import tpu_sc as plsc

N = 10000
E = 320000
D = 128
NW = 32
WPW = D // NW // 2
CHUNK = 16000
N_CHUNKS = E // CHUNK
GROUPS = CHUNK // 16
NB = 4

_MINF2 = jnp.int32(0xFF80FF80 - (1 << 32))


def _ab_body(feat_ref, wt_ref, wp_ref, bt_ref, bp_ref, a_ref, b_ref):
    x = feat_ref[...]
    wt = wt_ref[...]
    a_ref[...] = jnp.dot(x, wt, preferred_element_type=jnp.float32)
    b_ref[...] = (
        jnp.dot(x, wp_ref[...] - wt, preferred_element_type=jnp.float32)
        + bt_ref[...] + bp_ref[...]
    ).astype(jnp.bfloat16)


def _compute_ab(feat, wtT, wpT, bt, bp):
    grid = 10
    blk = N // grid
    return pl.pallas_call(
        _ab_body,
        grid=(grid,),
        in_specs=[
            pl.BlockSpec((blk, D), lambda i: (i, 0)),
            pl.BlockSpec((D, D), lambda i: (0, 0)),
            pl.BlockSpec((D, D), lambda i: (0, 0)),
            pl.BlockSpec((1, D), lambda i: (0, 0)),
            pl.BlockSpec((1, D), lambda i: (0, 0)),
        ],
        out_specs=[
            pl.BlockSpec((blk, D), lambda i: (i, 0)),
            pl.BlockSpec((blk, D), lambda i: (i, 0)),
        ],
        out_shape=[
            jax.ShapeDtypeStruct((N, D), jnp.float32),
            jax.ShapeDtypeStruct((N, D), jnp.bfloat16),
        ],
    )(feat, wtT, wpT, bt, bp)


def _sc_body(b_hbm, src_hbm, dst_hbm, m_hbm,
             bv0, bv1, mv0, mv1,
             sv0, dv0, sv1, dv1, sem0, sem1):
    cid = lax.axis_index("c")
    sid = lax.axis_index("s")
    wid = sid * 2 + cid
    bvs = [bv0, bv1]
    mvs = [mv0, mv1]
    svs = [sv0, sv1]
    dvs = [dv0, dv1]
    sems = [sem0, sem1]

    def start_copy(k, buf):
        sl = pl.ds(k * CHUNK, CHUNK)
        pltpu.make_async_copy(src_hbm.at[sl], svs[buf], sems[buf]).start()
        pltpu.make_async_copy(dst_hbm.at[sl], dvs[buf], sems[buf]).start()

    def wait_copy(k, buf):
        sl = pl.ds(k * CHUNK, CHUNK)
        pltpu.make_async_copy(src_hbm.at[sl], svs[buf], sems[buf]).wait()
        pltpu.make_async_copy(dst_hbm.at[sl], dvs[buf], sems[buf]).wait()

    for u in range(WPW):
        pltpu.sync_copy(b_hbm.at[wid, u], bvs[u])

    minf = jnp.full((16,), _MINF2, jnp.int32)

    def init(i, carry):
        for u in range(WPW):
            mvs[u][pl.ds(i * 16, 16)] = minf
        return carry

    lax.fori_loop(0, N // 16, init, 0, unroll=4)

    def _bf(x_i32):
        return plsc.bitcast(x_i32, jnp.bfloat16)

    def _i32(x_bf):
        return plsc.bitcast(x_bf, jnp.int32)

    def process(sv, dv):
        def block(t, c2):
            base = t * (16 * NB)
            d16s = [dv[pl.ds(base + 16 * i, 16)] for i in range(NB)]
            s16s = [sv[pl.ds(base + 16 * i, 16)] for i in range(NB)]
            valss = [
                [_bf(plsc.load_gather(bvs[u], [s16s[i]])) for u in range(WPW)]
                for i in range(NB)
            ]
            lasts = [plsc.scan_count(d16s[i])[1] for i in range(NB)]
            for i in range(NB):
                curs = [
                    _bf(plsc.load_gather(mvs[u], [d16s[i]]))
                    for u in range(WPW)
                ]
                news = [
                    jnp.maximum(curs[u], valss[i][u]) for u in range(WPW)
                ]
                for u in range(WPW):
                    plsc.store_scatter(
                        mvs[u], [d16s[i]], _i32(news[u]), mask=lasts[i]
                    )

            rems = [jnp.logical_not(lasts[i]) for i in range(NB)]
            any_rem = rems[0]
            for i in range(1, NB):
                any_rem = jnp.logical_or(any_rem, rems[i])

            @pl.when(jnp.any(any_rem))
            def _slow():
                for i in range(NB):
                    @pl.when(jnp.any(rems[i]))
                    def _g(i=i):
                        def round_(rem):
                            _, last2 = plsc.scan_count(d16s[i], mask=rem)
                            for u in range(WPW):
                                cur = _bf(
                                    plsc.load_gather(mvs[u], [d16s[i]])
                                )
                                plsc.store_scatter(
                                    mvs[u], [d16s[i]],
                                    _i32(jnp.maximum(cur, valss[i][u])),
                                    mask=last2,
                                )
                            return jnp.logical_and(
                                rem, jnp.logical_not(last2)
                            )

                        lax.while_loop(
                            lambda r: jnp.any(r), round_, rems[i]
                        )

            return c2

        lax.fori_loop(0, GROUPS // NB, block, 0)

    start_copy(0, 0)

    def pair(p, carry):
        start_copy(2 * p + 1, 1)
        wait_copy(2 * p, 0)
        process(svs[0], dvs[0])

        @pl.when(p < N_CHUNKS // 2 - 1)
        def _prefetch():
            start_copy(2 * p + 2, 0)

        wait_copy(2 * p + 1, 1)
        process(svs[1], dvs[1])
        return carry

    lax.fori_loop(0, N_CHUNKS // 2, pair, 0)

    for u in range(WPW):
        pltpu.sync_copy(mvs[u], m_hbm.at[wid, u])


def _sc_scatter_max(b_r, src, dst):
    mesh = plsc.VectorSubcoreMesh(core_axis_name="c", subcore_axis_name="s")
    f = functools.partial(
        pl.kernel,
        out_type=jax.ShapeDtypeStruct((NW, WPW, N), jnp.int32),
        mesh=mesh,
        scratch_types=(
            [pltpu.VMEM((N,), jnp.int32) for _ in range(2 * WPW)]
            + [pltpu.VMEM((CHUNK,), jnp.int32) for _ in range(4)]
            + [pltpu.SemaphoreType.DMA for _ in range(2)]
        ),
        compiler_params=pltpu.CompilerParams(needs_layout_passes=False),
    )(_sc_body)
    return f(b_r, src, dst)


def _combine_body(a_ref, m_ref, o_ref):
    m = m_ref[...].astype(jnp.float32)
    o_ref[...] = jnp.where(jnp.isfinite(m), a_ref[...] + m, 0.0)


def _combine(a, m):
    grid = 10
    blk = N // grid
    return pl.pallas_call(
        _combine_body,
        grid=(grid,),
        in_specs=[
            pl.BlockSpec((blk, D), lambda i: (i, 0)),
            pl.BlockSpec((blk, D), lambda i: (i, 0)),
        ],
        out_specs=pl.BlockSpec((blk, D), lambda i: (i, 0)),
        out_shape=jax.ShapeDtypeStruct((N, D), jnp.float32),
    )(a, m)


def kernel(feat, edge_index, W_theta, b_theta, W_phi, b_phi):
    src = edge_index[0].astype(jnp.int32)
    dst = edge_index[1].astype(jnp.int32)
    a, b = _compute_ab(
        feat,
        W_theta.T,
        W_phi.T,
        b_theta.reshape(1, D),
        b_phi.reshape(1, D),
    )
    b_p = lax.bitcast_convert_type(b.reshape(N, D // 2, 2), jnp.int32)
    b_r = b_p.reshape(N, NW, WPW).transpose(1, 2, 0)
    m_r = _sc_scatter_max(b_r, src, dst)
    m_p = m_r.transpose(2, 0, 1).reshape(N, D // 2)
    m = lax.bitcast_convert_type(m_p, jnp.bfloat16).reshape(N, D)
    return _combine(a, m)


# speedup vs baseline: 8.7739x; 1.0237x over previous
import functools

import jax
import jax.numpy as jnp
from jax import lax
from jax.experimental import pallas as pl
from jax.experimental.pallas import tpu as pltpu
from jax.experimental.pallas import tpu_sc as plsc

N = 10000
E = 320000
D = 128
NW = 32
WPW = D // NW // 2
CHUNK = 16000
N_CHUNKS = E // CHUNK
GROUPS = CHUNK // 16
NB = 8

_MINF2 = jnp.int32(0xFF80FF80 - (1 << 32))


def _ab_body(feat_ref, wt_ref, wp_ref, bt_ref, bp_ref, a_ref, b_ref):
    x = feat_ref[...]
    wt = wt_ref[...]
    a_ref[...] = jnp.dot(x, wt, preferred_element_type=jnp.float32)
    b_ref[...] = (
        jnp.dot(x, wp_ref[...] - wt, preferred_element_type=jnp.float32)
        + bt_ref[...] + bp_ref[...]
    ).astype(jnp.bfloat16)


def _compute_ab(feat, wtT, wpT, bt, bp):
    grid = 10
    blk = N // grid
    return pl.pallas_call(
        _ab_body,
        grid=(grid,),
        in_specs=[
            pl.BlockSpec((blk, D), lambda i: (i, 0)),
            pl.BlockSpec((D, D), lambda i: (0, 0)),
            pl.BlockSpec((D, D), lambda i: (0, 0)),
            pl.BlockSpec((1, D), lambda i: (0, 0)),
            pl.BlockSpec((1, D), lambda i: (0, 0)),
        ],
        out_specs=[
            pl.BlockSpec((blk, D), lambda i: (i, 0)),
            pl.BlockSpec((blk, D), lambda i: (i, 0)),
        ],
        out_shape=[
            jax.ShapeDtypeStruct((N, D), jnp.float32),
            jax.ShapeDtypeStruct((N, D), jnp.bfloat16),
        ],
    )(feat, wtT, wpT, bt, bp)


def _sc_body(b_hbm, src_hbm, dst_hbm, m_hbm,
             bv0, bv1, mv0, mv1,
             sv0, dv0, sv1, dv1, sem0, sem1):
    cid = lax.axis_index("c")
    sid = lax.axis_index("s")
    wid = sid * 2 + cid
    bvs = [bv0, bv1]
    mvs = [mv0, mv1]
    svs = [sv0, sv1]
    dvs = [dv0, dv1]
    sems = [sem0, sem1]

    def start_copy(k, buf):
        sl = pl.ds(k * CHUNK, CHUNK)
        pltpu.make_async_copy(src_hbm.at[sl], svs[buf], sems[buf]).start()
        pltpu.make_async_copy(dst_hbm.at[sl], dvs[buf], sems[buf]).start()

    def wait_copy(k, buf):
        sl = pl.ds(k * CHUNK, CHUNK)
        pltpu.make_async_copy(src_hbm.at[sl], svs[buf], sems[buf]).wait()
        pltpu.make_async_copy(dst_hbm.at[sl], dvs[buf], sems[buf]).wait()

    for u in range(WPW):
        pltpu.sync_copy(b_hbm.at[wid, u], bvs[u])

    minf = jnp.full((16,), _MINF2, jnp.int32)

    def init(i, carry):
        for u in range(WPW):
            mvs[u][pl.ds(i * 16, 16)] = minf
        return carry

    lax.fori_loop(0, N // 16, init, 0, unroll=4)

    def _bf(x_i32):
        return plsc.bitcast(x_i32, jnp.bfloat16)

    def _i32(x_bf):
        return plsc.bitcast(x_bf, jnp.int32)

    def process(sv, dv):
        def block(t, c2):
            base = t * (16 * NB)
            d16s = [dv[pl.ds(base + 16 * i, 16)] for i in range(NB)]
            s16s = [sv[pl.ds(base + 16 * i, 16)] for i in range(NB)]
            valss = [
                [_bf(plsc.load_gather(bvs[u], [s16s[i]])) for u in range(WPW)]
                for i in range(NB)
            ]
            lasts = [plsc.scan_count(d16s[i])[1] for i in range(NB)]
            for i in range(NB):
                curs = [
                    _bf(plsc.load_gather(mvs[u], [d16s[i]]))
                    for u in range(WPW)
                ]
                news = [
                    jnp.maximum(curs[u], valss[i][u]) for u in range(WPW)
                ]
                for u in range(WPW):
                    plsc.store_scatter(
                        mvs[u], [d16s[i]], _i32(news[u]), mask=lasts[i]
                    )

            rems = [jnp.logical_not(lasts[i]) for i in range(NB)]
            any_rem = rems[0]
            for i in range(1, NB):
                any_rem = jnp.logical_or(any_rem, rems[i])

            @pl.when(jnp.any(any_rem))
            def _slow():
                for i in range(NB):
                    @pl.when(jnp.any(rems[i]))
                    def _g(i=i):
                        def round_(rem):
                            _, last2 = plsc.scan_count(d16s[i], mask=rem)
                            for u in range(WPW):
                                cur = _bf(
                                    plsc.load_gather(mvs[u], [d16s[i]])
                                )
                                plsc.store_scatter(
                                    mvs[u], [d16s[i]],
                                    _i32(jnp.maximum(cur, valss[i][u])),
                                    mask=last2,
                                )
                            return jnp.logical_and(
                                rem, jnp.logical_not(last2)
                            )

                        lax.while_loop(
                            lambda r: jnp.any(r), round_, rems[i]
                        )

            return c2

        lax.fori_loop(0, GROUPS // NB, block, 0)

    start_copy(0, 0)

    def pair(p, carry):
        start_copy(2 * p + 1, 1)
        wait_copy(2 * p, 0)
        process(svs[0], dvs[0])

        @pl.when(p < N_CHUNKS // 2 - 1)
        def _prefetch():
            start_copy(2 * p + 2, 0)

        wait_copy(2 * p + 1, 1)
        process(svs[1], dvs[1])
        return carry

    lax.fori_loop(0, N_CHUNKS // 2, pair, 0)

    for u in range(WPW):
        pltpu.sync_copy(mvs[u], m_hbm.at[wid, u])


def _sc_scatter_max(b_r, src, dst):
    mesh = plsc.VectorSubcoreMesh(core_axis_name="c", subcore_axis_name="s")
    f = functools.partial(
        pl.kernel,
        out_type=jax.ShapeDtypeStruct((NW, WPW, N), jnp.int32),
        mesh=mesh,
        scratch_types=(
            [pltpu.VMEM((N,), jnp.int32) for _ in range(2 * WPW)]
            + [pltpu.VMEM((CHUNK,), jnp.int32) for _ in range(4)]
            + [pltpu.SemaphoreType.DMA for _ in range(2)]
        ),
        compiler_params=pltpu.CompilerParams(needs_layout_passes=False),
    )(_sc_body)
    return f(b_r, src, dst)


def _combine_body(a_ref, m_ref, o_ref):
    m = m_ref[...].astype(jnp.float32)
    o_ref[...] = jnp.where(jnp.isfinite(m), a_ref[...] + m, 0.0)


def _combine(a, m):
    grid = 10
    blk = N // grid
    return pl.pallas_call(
        _combine_body,
        grid=(grid,),
        in_specs=[
            pl.BlockSpec((blk, D), lambda i: (i, 0)),
            pl.BlockSpec((blk, D), lambda i: (i, 0)),
        ],
        out_specs=pl.BlockSpec((blk, D), lambda i: (i, 0)),
        out_shape=jax.ShapeDtypeStruct((N, D), jnp.float32),
    )(a, m)


def kernel(feat, edge_index, W_theta, b_theta, W_phi, b_phi):
    src = edge_index[0].astype(jnp.int32)
    dst = edge_index[1].astype(jnp.int32)
    a, b = _compute_ab(
        feat,
        W_theta.T,
        W_phi.T,
        b_theta.reshape(1, D),
        b_phi.reshape(1, D),
    )
    b_p = lax.bitcast_convert_type(b.reshape(N, D // 2, 2), jnp.int32)
    b_r = b_p.reshape(N, NW, WPW).transpose(1, 2, 0)
    m_r = _sc_scatter_max(b_r, src, dst)
    m_p = m_r.transpose(2, 0, 1).reshape(N, D // 2)
    m = lax.bitcast_convert_type(m_p, jnp.bfloat16).reshape(N, D)
    return _combine(a, m)


# speedup vs baseline: 9.8193x; 1.1192x over previous
import functools

import jax
import jax.numpy as jnp
from jax import lax
from jax.experimental import pallas as pl
from jax.experimental.pallas import tpu as pltpu
from jax.experimental.pallas import tpu_sc as plsc

N = 10000
E = 320000
D = 128
NW = 32
WPW = D // NW // 2
CHUNK = 6400
N_CHUNKS = E // CHUNK
GROUPS = CHUNK // 16
NB = 8
NPAR = 2

_MINF2 = jnp.int32(0xFF80FF80 - (1 << 32))


def _ab_body(feat_ref, wt_ref, wp_ref, bt_ref, bp_ref, a_ref, b_ref):
    x = feat_ref[...]
    wt = wt_ref[...]
    a_ref[...] = jnp.dot(x, wt, preferred_element_type=jnp.float32)
    b_ref[...] = (
        jnp.dot(x, wp_ref[...] - wt, preferred_element_type=jnp.float32)
        + bt_ref[...] + bp_ref[...]
    ).astype(jnp.bfloat16)


def _compute_ab(feat, wtT, wpT, bt, bp):
    grid = 10
    blk = N // grid
    return pl.pallas_call(
        _ab_body,
        grid=(grid,),
        in_specs=[
            pl.BlockSpec((blk, D), lambda i: (i, 0)),
            pl.BlockSpec((D, D), lambda i: (0, 0)),
            pl.BlockSpec((D, D), lambda i: (0, 0)),
            pl.BlockSpec((1, D), lambda i: (0, 0)),
            pl.BlockSpec((1, D), lambda i: (0, 0)),
        ],
        out_specs=[
            pl.BlockSpec((blk, D), lambda i: (i, 0)),
            pl.BlockSpec((blk, D), lambda i: (i, 0)),
        ],
        out_shape=[
            jax.ShapeDtypeStruct((N, D), jnp.float32),
            jax.ShapeDtypeStruct((N, D), jnp.bfloat16),
        ],
    )(feat, wtT, wpT, bt, bp)


def _sc_body(b_hbm, src_hbm, dst_hbm, m_hbm,
             bv0, bv1, me0, me1, mo0, mo1,
             sv0, dv0, sv1, dv1, sem0, sem1):
    cid = lax.axis_index("c")
    sid = lax.axis_index("s")
    wid = sid * 2 + cid
    bvs = [bv0, bv1]
    accs = [[me0, me1], [mo0, mo1]]
    svs = [sv0, sv1]
    dvs = [dv0, dv1]
    sems = [sem0, sem1]

    def start_copy(k, buf):
        sl = pl.ds(k * CHUNK, CHUNK)
        pltpu.make_async_copy(src_hbm.at[sl], svs[buf], sems[buf]).start()
        pltpu.make_async_copy(dst_hbm.at[sl], dvs[buf], sems[buf]).start()

    def wait_copy(k, buf):
        sl = pl.ds(k * CHUNK, CHUNK)
        pltpu.make_async_copy(src_hbm.at[sl], svs[buf], sems[buf]).wait()
        pltpu.make_async_copy(dst_hbm.at[sl], dvs[buf], sems[buf]).wait()

    for u in range(WPW):
        pltpu.sync_copy(b_hbm.at[wid, u], bvs[u])

    minf = jnp.full((16,), _MINF2, jnp.int32)

    def init(i, carry):
        for p in range(NPAR):
            for u in range(WPW):
                accs[p][u][pl.ds(i * 16, 16)] = minf
        return carry

    lax.fori_loop(0, N // 16, init, 0, unroll=4)

    def _bf(x_i32):
        return plsc.bitcast(x_i32, jnp.bfloat16)

    def _i32(x_bf):
        return plsc.bitcast(x_bf, jnp.int32)

    def process(sv, dv):
        def block(t, c2):
            base = t * (16 * NB)
            d16s = [dv[pl.ds(base + 16 * i, 16)] for i in range(NB)]
            s16s = [sv[pl.ds(base + 16 * i, 16)] for i in range(NB)]
            valss = [
                [_bf(plsc.load_gather(bvs[u], [s16s[i]])) for u in range(WPW)]
                for i in range(NB)
            ]
            lasts = [plsc.scan_count(d16s[i])[1] for i in range(NB)]
            for i0 in range(0, NB, NPAR):
                pidx = list(range(i0, i0 + NPAR))
                curs = {
                    i: [
                        _bf(plsc.load_gather(accs[i % NPAR][u], [d16s[i]]))
                        for u in range(WPW)
                    ]
                    for i in pidx
                }
                for i in pidx:
                    for u in range(WPW):
                        plsc.store_scatter(
                            accs[i % NPAR][u], [d16s[i]],
                            _i32(jnp.maximum(curs[i][u], valss[i][u])),
                            mask=lasts[i],
                        )

            rems = [jnp.logical_not(lasts[i]) for i in range(NB)]
            any_rem = rems[0]
            for i in range(1, NB):
                any_rem = jnp.logical_or(any_rem, rems[i])

            @pl.when(jnp.any(any_rem))
            def _slow():
                for i in range(NB):
                    @pl.when(jnp.any(rems[i]))
                    def _g(i=i):
                        def round_(rem):
                            _, last2 = plsc.scan_count(d16s[i], mask=rem)
                            macc = accs[i % NPAR]
                            for u in range(WPW):
                                cur = _bf(
                                    plsc.load_gather(macc[u], [d16s[i]])
                                )
                                plsc.store_scatter(
                                    macc[u], [d16s[i]],
                                    _i32(jnp.maximum(cur, valss[i][u])),
                                    mask=last2,
                                )
                            return jnp.logical_and(
                                rem, jnp.logical_not(last2)
                            )

                        lax.while_loop(
                            lambda r: jnp.any(r), round_, rems[i]
                        )

            return c2

        lax.fori_loop(0, GROUPS // NB, block, 0)

    start_copy(0, 0)

    def pair(p, carry):
        start_copy(2 * p + 1, 1)
        wait_copy(2 * p, 0)
        process(svs[0], dvs[0])

        @pl.when(p < N_CHUNKS // 2 - 1)
        def _prefetch():
            start_copy(2 * p + 2, 0)

        wait_copy(2 * p + 1, 1)
        process(svs[1], dvs[1])
        return carry

    lax.fori_loop(0, N_CHUNKS // 2, pair, 0)

    def merge(i, carry):
        sl = pl.ds(i * 16, 16)
        for u in range(WPW):
            m = jnp.maximum(_bf(accs[0][u][sl]), _bf(accs[1][u][sl]))
            accs[0][u][sl] = _i32(m)
        return carry

    lax.fori_loop(0, N // 16, merge, 0, unroll=4)

    for u in range(WPW):
        pltpu.sync_copy(accs[0][u], m_hbm.at[wid, u])


def _sc_scatter_max(b_r, src, dst):
    mesh = plsc.VectorSubcoreMesh(core_axis_name="c", subcore_axis_name="s")
    f = functools.partial(
        pl.kernel,
        out_type=jax.ShapeDtypeStruct((NW, WPW, N), jnp.int32),
        mesh=mesh,
        scratch_types=(
            [pltpu.VMEM((N,), jnp.int32) for _ in range((1 + NPAR) * WPW)]
            + [pltpu.VMEM((CHUNK,), jnp.int32) for _ in range(4)]
            + [pltpu.SemaphoreType.DMA for _ in range(2)]
        ),
        compiler_params=pltpu.CompilerParams(needs_layout_passes=False),
    )(_sc_body)
    return f(b_r, src, dst)


def _combine_body(a_ref, m_ref, o_ref):
    m = m_ref[...].astype(jnp.float32)
    o_ref[...] = jnp.where(jnp.isfinite(m), a_ref[...] + m, 0.0)


def _combine(a, m):
    grid = 10
    blk = N // grid
    return pl.pallas_call(
        _combine_body,
        grid=(grid,),
        in_specs=[
            pl.BlockSpec((blk, D), lambda i: (i, 0)),
            pl.BlockSpec((blk, D), lambda i: (i, 0)),
        ],
        out_specs=pl.BlockSpec((blk, D), lambda i: (i, 0)),
        out_shape=jax.ShapeDtypeStruct((N, D), jnp.float32),
    )(a, m)


def kernel(feat, edge_index, W_theta, b_theta, W_phi, b_phi):
    src = edge_index[0].astype(jnp.int32)
    dst = edge_index[1].astype(jnp.int32)
    a, b = _compute_ab(
        feat,
        W_theta.T,
        W_phi.T,
        b_theta.reshape(1, D),
        b_phi.reshape(1, D),
    )
    b_p = lax.bitcast_convert_type(b.reshape(N, D // 2, 2), jnp.int32)
    b_r = b_p.reshape(N, NW, WPW).transpose(1, 2, 0)
    m_r = _sc_scatter_max(b_r, src, dst)
    m_p = m_r.transpose(2, 0, 1).reshape(N, D // 2)
    m = lax.bitcast_convert_type(m_p, jnp.bfloat16).reshape(N, D)
    return _combine(a, m)


# speedup vs baseline: 9.9786x; 1.0162x over previous
import functools

import jax
import jax.numpy as jnp
from jax import lax
from jax.experimental import pallas as pl
from jax.experimental.pallas import tpu as pltpu
from jax.experimental.pallas import tpu_sc as plsc

N = 10000
E = 320000
D = 128
NW = 32
WPW = D // NW // 2
CHUNK = 6400
N_CHUNKS = E // CHUNK
GROUPS = CHUNK // 16
NB = 8
NPAR = 2

_MINF2 = 0xFF80FF80 - (1 << 32)


def _ab_body(feat_ref, wt_ref, wp_ref, bt_ref, bp_ref, a_ref, b_ref):
    x = feat_ref[...]
    wt = wt_ref[...]
    a_ref[...] = jnp.dot(x, wt, preferred_element_type=jnp.float32)
    b_ref[...] = (
        jnp.dot(x, wp_ref[...] - wt, preferred_element_type=jnp.float32)
        + bt_ref[...] + bp_ref[...]
    ).astype(jnp.bfloat16)


def _compute_ab(feat, wtT, wpT, bt, bp):
    grid = 10
    blk = N // grid
    return pl.pallas_call(
        _ab_body,
        grid=(grid,),
        in_specs=[
            pl.BlockSpec((blk, D), lambda i: (i, 0)),
            pl.BlockSpec((D, D), lambda i: (0, 0)),
            pl.BlockSpec((D, D), lambda i: (0, 0)),
            pl.BlockSpec((1, D), lambda i: (0, 0)),
            pl.BlockSpec((1, D), lambda i: (0, 0)),
        ],
        out_specs=[
            pl.BlockSpec((blk, D), lambda i: (i, 0)),
            pl.BlockSpec((blk, D), lambda i: (i, 0)),
        ],
        out_shape=[
            jax.ShapeDtypeStruct((N, D), jnp.float32),
            jax.ShapeDtypeStruct((N, D), jnp.bfloat16),
        ],
    )(feat, wtT, wpT, bt, bp)


def _sc_body(b_hbm, edge_hbm, m_hbm,
             bv0, bv1, me0, me1, mo0, mo1,
             ev0, ev1, sem0, sem1):
    cid = lax.axis_index("c")
    sid = lax.axis_index("s")
    wid = sid * 2 + cid
    bvs = [bv0, bv1]
    accs = [[me0, me1], [mo0, mo1]]
    evs = [ev0, ev1]
    sems = [sem0, sem1]

    def start_copy(k, buf):
        sl = pl.ds(k * CHUNK, CHUNK)
        pltpu.make_async_copy(edge_hbm.at[sl], evs[buf], sems[buf]).start()

    def wait_copy(k, buf):
        sl = pl.ds(k * CHUNK, CHUNK)
        pltpu.make_async_copy(edge_hbm.at[sl], evs[buf], sems[buf]).wait()

    for u in range(WPW):
        pltpu.sync_copy(b_hbm.at[wid, u], bvs[u])

    minf = jnp.full((16,), _MINF2, jnp.int32)

    def init(i, carry):
        for p in range(NPAR):
            for u in range(WPW):
                accs[p][u][pl.ds(i * 16, 16)] = minf
        return carry

    lax.fori_loop(0, N // 16, init, 0, unroll=4)

    def _bf(x_i32):
        return plsc.bitcast(x_i32, jnp.bfloat16)

    def _i32(x_bf):
        return plsc.bitcast(x_bf, jnp.int32)

    def process(ev):
        def block(t, c2):
            base = t * (16 * NB)
            e16s = [ev[pl.ds(base + 16 * i, 16)] for i in range(NB)]
            d16s = [lax.shift_right_logical(e16s[i], 16) for i in range(NB)]
            s16s = [jnp.bitwise_and(e16s[i], 0xFFFF) for i in range(NB)]
            valss = [
                [_bf(plsc.load_gather(bvs[u], [s16s[i]])) for u in range(WPW)]
                for i in range(NB)
            ]
            lasts = [plsc.scan_count(d16s[i])[1] for i in range(NB)]
            for i0 in range(0, NB, NPAR):
                pidx = list(range(i0, i0 + NPAR))
                curs = {
                    i: [
                        _bf(plsc.load_gather(accs[i % NPAR][u], [d16s[i]]))
                        for u in range(WPW)
                    ]
                    for i in pidx
                }
                for i in pidx:
                    for u in range(WPW):
                        plsc.store_scatter(
                            accs[i % NPAR][u], [d16s[i]],
                            _i32(jnp.maximum(curs[i][u], valss[i][u])),
                            mask=lasts[i],
                        )

            rems = [jnp.logical_not(lasts[i]) for i in range(NB)]
            any_rem = rems[0]
            for i in range(1, NB):
                any_rem = jnp.logical_or(any_rem, rems[i])

            @pl.when(jnp.any(any_rem))
            def _slow():
                for i in range(NB):
                    @pl.when(jnp.any(rems[i]))
                    def _g(i=i):
                        def round_(rem):
                            _, last2 = plsc.scan_count(d16s[i], mask=rem)
                            macc = accs[i % NPAR]
                            for u in range(WPW):
                                cur = _bf(
                                    plsc.load_gather(macc[u], [d16s[i]])
                                )
                                plsc.store_scatter(
                                    macc[u], [d16s[i]],
                                    _i32(jnp.maximum(cur, valss[i][u])),
                                    mask=last2,
                                )
                            return jnp.logical_and(
                                rem, jnp.logical_not(last2)
                            )

                        lax.while_loop(
                            lambda r: jnp.any(r), round_, rems[i]
                        )

            return c2

        lax.fori_loop(0, GROUPS // NB, block, 0)

    start_copy(0, 0)

    def pair(p, carry):
        start_copy(2 * p + 1, 1)
        wait_copy(2 * p, 0)
        process(evs[0])

        @pl.when(p < N_CHUNKS // 2 - 1)
        def _prefetch():
            start_copy(2 * p + 2, 0)

        wait_copy(2 * p + 1, 1)
        process(evs[1])
        return carry

    lax.fori_loop(0, N_CHUNKS // 2, pair, 0)

    def merge(i, carry):
        sl = pl.ds(i * 16, 16)
        for u in range(WPW):
            m = jnp.maximum(_bf(accs[0][u][sl]), _bf(accs[1][u][sl]))
            accs[0][u][sl] = _i32(m)
        return carry

    lax.fori_loop(0, N // 16, merge, 0, unroll=4)

    for u in range(WPW):
        pltpu.sync_copy(accs[0][u], m_hbm.at[wid, u])


def _sc_scatter_max(b_r, edges):
    mesh = plsc.VectorSubcoreMesh(core_axis_name="c", subcore_axis_name="s")
    f = functools.partial(
        pl.kernel,
        out_type=jax.ShapeDtypeStruct((NW, WPW, N), jnp.int32),
        mesh=mesh,
        scratch_types=(
            [pltpu.VMEM((N,), jnp.int32) for _ in range((1 + NPAR) * WPW)]
            + [pltpu.VMEM((CHUNK,), jnp.int32) for _ in range(2)]
            + [pltpu.SemaphoreType.DMA for _ in range(2)]
        ),
        compiler_params=pltpu.CompilerParams(needs_layout_passes=False),
    )(_sc_body)
    return f(b_r, edges)


def _combine_body(a_ref, m_ref, o_ref):
    m = m_ref[...].astype(jnp.float32)
    o_ref[...] = jnp.where(jnp.isfinite(m), a_ref[...] + m, 0.0)


def _combine(a, m):
    grid = 10
    blk = N // grid
    return pl.pallas_call(
        _combine_body,
        grid=(grid,),
        in_specs=[
            pl.BlockSpec((blk, D), lambda i: (i, 0)),
            pl.BlockSpec((blk, D), lambda i: (i, 0)),
        ],
        out_specs=pl.BlockSpec((blk, D), lambda i: (i, 0)),
        out_shape=jax.ShapeDtypeStruct((N, D), jnp.float32),
    )(a, m)


def kernel(feat, edge_index, W_theta, b_theta, W_phi, b_phi):
    src = edge_index[0].astype(jnp.int32)
    dst = edge_index[1].astype(jnp.int32)
    edges = jnp.bitwise_or(src, jnp.left_shift(dst, 16))
    a, b = _compute_ab(
        feat,
        W_theta.T,
        W_phi.T,
        b_theta.reshape(1, D),
        b_phi.reshape(1, D),
    )
    b_p = lax.bitcast_convert_type(b.reshape(N, D // 2, 2), jnp.int32)
    b_r = b_p.reshape(N, NW, WPW).transpose(1, 2, 0)
    m_r = _sc_scatter_max(b_r, edges)
    m_p = m_r.transpose(2, 0, 1).reshape(N, D // 2)
    m = lax.bitcast_convert_type(m_p, jnp.bfloat16).reshape(N, D)
    return _combine(a, m)


# speedup vs baseline: 10.6989x; 1.0722x over previous
import functools

import jax
import jax.numpy as jnp
from jax import lax
from jax.experimental import pallas as pl
from jax.experimental.pallas import tpu as pltpu
from jax.experimental.pallas import tpu_sc as plsc

N = 10000
E = 320000
D = 128
NW = 32
WPW = D // NW // 2
CHUNK = 6400
N_CHUNKS = E // CHUNK
GROUPS = CHUNK // 16
NB = 8
NPAR = 4

_MINF2 = 0xFF80FF80 - (1 << 32)


def _ab_body(feat_ref, wt_ref, wp_ref, bt_ref, bp_ref, a_ref, b_ref):
    x = feat_ref[...]
    wt = wt_ref[...]
    a_ref[...] = jnp.dot(x, wt, preferred_element_type=jnp.float32)
    b_ref[...] = (
        jnp.dot(x, wp_ref[...] - wt, preferred_element_type=jnp.float32)
        + bt_ref[...] + bp_ref[...]
    ).astype(jnp.bfloat16)


def _compute_ab(feat, wtT, wpT, bt, bp):
    grid = 10
    blk = N // grid
    return pl.pallas_call(
        _ab_body,
        grid=(grid,),
        in_specs=[
            pl.BlockSpec((blk, D), lambda i: (i, 0)),
            pl.BlockSpec((D, D), lambda i: (0, 0)),
            pl.BlockSpec((D, D), lambda i: (0, 0)),
            pl.BlockSpec((1, D), lambda i: (0, 0)),
            pl.BlockSpec((1, D), lambda i: (0, 0)),
        ],
        out_specs=[
            pl.BlockSpec((blk, D), lambda i: (i, 0)),
            pl.BlockSpec((blk, D), lambda i: (i, 0)),
        ],
        out_shape=[
            jax.ShapeDtypeStruct((N, D), jnp.float32),
            jax.ShapeDtypeStruct((N, D), jnp.bfloat16),
        ],
    )(feat, wtT, wpT, bt, bp)


def _sc_body(b_hbm, edge_hbm, m_hbm,
             bv0, bv1, ma0, ma1, mb0, mb1, mc0, mc1, md0, md1,
             ev0, ev1, sem0, sem1):
    cid = lax.axis_index("c")
    sid = lax.axis_index("s")
    wid = sid * 2 + cid
    bvs = [bv0, bv1]
    accs = [[ma0, ma1], [mb0, mb1], [mc0, mc1], [md0, md1]]
    evs = [ev0, ev1]
    sems = [sem0, sem1]

    def start_copy(k, buf):
        sl = pl.ds(k * CHUNK, CHUNK)
        pltpu.make_async_copy(edge_hbm.at[sl], evs[buf], sems[buf]).start()

    def wait_copy(k, buf):
        sl = pl.ds(k * CHUNK, CHUNK)
        pltpu.make_async_copy(edge_hbm.at[sl], evs[buf], sems[buf]).wait()

    for u in range(WPW):
        pltpu.sync_copy(b_hbm.at[wid, u], bvs[u])

    minf = jnp.full((16,), _MINF2, jnp.int32)

    def init(i, carry):
        for p in range(NPAR):
            for u in range(WPW):
                accs[p][u][pl.ds(i * 16, 16)] = minf
        return carry

    lax.fori_loop(0, N // 16, init, 0, unroll=4)

    def _bf(x_i32):
        return plsc.bitcast(x_i32, jnp.bfloat16)

    def _i32(x_bf):
        return plsc.bitcast(x_bf, jnp.int32)

    def process(ev):
        def block(t, c2):
            base = t * (16 * NB)
            e16s = [ev[pl.ds(base + 16 * i, 16)] for i in range(NB)]
            d16s = [lax.shift_right_logical(e16s[i], 16) for i in range(NB)]
            s16s = [jnp.bitwise_and(e16s[i], 0xFFFF) for i in range(NB)]
            valss = [
                [_bf(plsc.load_gather(bvs[u], [s16s[i]])) for u in range(WPW)]
                for i in range(NB)
            ]
            lasts = [plsc.scan_count(d16s[i])[1] for i in range(NB)]
            for i0 in range(0, NB, NPAR):
                pidx = list(range(i0, i0 + NPAR))
                curs = {
                    i: [
                        _bf(plsc.load_gather(accs[i % NPAR][u], [d16s[i]]))
                        for u in range(WPW)
                    ]
                    for i in pidx
                }
                for i in pidx:
                    for u in range(WPW):
                        plsc.store_scatter(
                            accs[i % NPAR][u], [d16s[i]],
                            _i32(jnp.maximum(curs[i][u], valss[i][u])),
                            mask=lasts[i],
                        )

            rems = [jnp.logical_not(lasts[i]) for i in range(NB)]
            any_rem = rems[0]
            for i in range(1, NB):
                any_rem = jnp.logical_or(any_rem, rems[i])

            @pl.when(jnp.any(any_rem))
            def _slow():
                for i in range(NB):
                    @pl.when(jnp.any(rems[i]))
                    def _g(i=i):
                        def round_(rem):
                            _, last2 = plsc.scan_count(d16s[i], mask=rem)
                            macc = accs[i % NPAR]
                            for u in range(WPW):
                                cur = _bf(
                                    plsc.load_gather(macc[u], [d16s[i]])
                                )
                                plsc.store_scatter(
                                    macc[u], [d16s[i]],
                                    _i32(jnp.maximum(cur, valss[i][u])),
                                    mask=last2,
                                )
                            return jnp.logical_and(
                                rem, jnp.logical_not(last2)
                            )

                        lax.while_loop(
                            lambda r: jnp.any(r), round_, rems[i]
                        )

            return c2

        lax.fori_loop(0, GROUPS // NB, block, 0)

    start_copy(0, 0)

    def pair(p, carry):
        start_copy(2 * p + 1, 1)
        wait_copy(2 * p, 0)
        process(evs[0])

        @pl.when(p < N_CHUNKS // 2 - 1)
        def _prefetch():
            start_copy(2 * p + 2, 0)

        wait_copy(2 * p + 1, 1)
        process(evs[1])
        return carry

    lax.fori_loop(0, N_CHUNKS // 2, pair, 0)

    def merge(i, carry):
        sl = pl.ds(i * 16, 16)
        for u in range(WPW):
            m01 = jnp.maximum(_bf(accs[0][u][sl]), _bf(accs[1][u][sl]))
            m23 = jnp.maximum(_bf(accs[2][u][sl]), _bf(accs[3][u][sl]))
            accs[0][u][sl] = _i32(jnp.maximum(m01, m23))
        return carry

    lax.fori_loop(0, N // 16, merge, 0, unroll=4)

    for u in range(WPW):
        pltpu.sync_copy(accs[0][u], m_hbm.at[wid, u])


def _sc_scatter_max(b_r, edges):
    mesh = plsc.VectorSubcoreMesh(core_axis_name="c", subcore_axis_name="s")
    f = functools.partial(
        pl.kernel,
        out_type=jax.ShapeDtypeStruct((NW, WPW, N), jnp.int32),
        mesh=mesh,
        scratch_types=(
            [pltpu.VMEM((N,), jnp.int32) for _ in range((1 + NPAR) * WPW)]
            + [pltpu.VMEM((CHUNK,), jnp.int32) for _ in range(2)]
            + [pltpu.SemaphoreType.DMA for _ in range(2)]
        ),
        compiler_params=pltpu.CompilerParams(needs_layout_passes=False),
    )(_sc_body)
    return f(b_r, edges)


def _combine_body(a_ref, m_ref, o_ref):
    m = m_ref[...].astype(jnp.float32)
    o_ref[...] = jnp.where(jnp.isfinite(m), a_ref[...] + m, 0.0)


def _combine(a, m):
    grid = 10
    blk = N // grid
    return pl.pallas_call(
        _combine_body,
        grid=(grid,),
        in_specs=[
            pl.BlockSpec((blk, D), lambda i: (i, 0)),
            pl.BlockSpec((blk, D), lambda i: (i, 0)),
        ],
        out_specs=pl.BlockSpec((blk, D), lambda i: (i, 0)),
        out_shape=jax.ShapeDtypeStruct((N, D), jnp.float32),
    )(a, m)


def kernel(feat, edge_index, W_theta, b_theta, W_phi, b_phi):
    src = edge_index[0].astype(jnp.int32)
    dst = edge_index[1].astype(jnp.int32)
    edges = jnp.bitwise_or(src, jnp.left_shift(dst, 16))
    a, b = _compute_ab(
        feat,
        W_theta.T,
        W_phi.T,
        b_theta.reshape(1, D),
        b_phi.reshape(1, D),
    )
    b_p = lax.bitcast_convert_type(b.reshape(N, D // 2, 2), jnp.int32)
    b_r = b_p.reshape(N, NW, WPW).transpose(1, 2, 0)
    m_r = _sc_scatter_max(b_r, edges)
    m_p = m_r.transpose(2, 0, 1).reshape(N, D // 2)
    m = lax.bitcast_convert_type(m_p, jnp.bfloat16).reshape(N, D)
    return _combine(a, m)
